# baseline (speedup 1.0000x reference)
import functools

import jax
import jax.numpy as jnp
from jax import lax
from jax.experimental import pallas as pl
from jax.experimental.pallas import tpu as pltpu
from jax.experimental.pallas import tpu_sc as plsc

_MS = 512.0
_H = 256
_W = 256
_NB = 50
_SCALE = 80.0
_WIN = 56
_K = 5
_CW = 128


def _target_kernel(in1, bpf, ints, out_cls, out_loc,
                   bminx, bminy, bmaxx, bmaxy, a1s, best):
    f32 = jnp.float32
    i32 = jnp.int32
    iotax = lax.broadcasted_iota(i32, (_H, _W), 1).astype(f32)
    iotay = lax.broadcasted_iota(i32, (_H, _W), 0).astype(f32)
    refx = iotax * (_MS / _W) + (_MS / _W / 2.0)
    refy = iotay * (_MS / _H) + (_MS / _H / 2.0)
    p0 = in1[0, 0, :, :]
    p1 = in1[0, 1, :, :]
    p2 = in1[0, 2, :, :]
    p3 = in1[0, 3, :, :]
    x1 = p0 * _SCALE + refx
    y1 = p1 * _SCALE + refy
    x2 = p2 * _SCALE + refx
    y2 = p3 * _SCALE + refy
    w = x2 - x1
    h = y2 - y1
    cx = x1 + w / 2.0
    cy = y1 + h / 2.0
    bminx[:, :] = cx - w / 2.0
    bmaxx[:, :] = cx + w / 2.0
    bminy[:, :] = cy - h / 2.0
    bmaxy[:, :] = cy + h / 2.0
    a1s[:, :] = w * h
    best[:, :] = jnp.zeros((_H, _W), f32)
    out_cls[0, 0, :, :] = jnp.ones((_H, _W), f32)
    out_cls[0, 1, :, :] = jnp.zeros((_H, _W), f32)
    for c in range(5):
        out_loc[0, c, :, :] = jnp.ones((_H, _W), f32)

    jgc = lax.broadcasted_iota(i32, (_WIN, _CW), 1).astype(f32)
    ig0 = lax.broadcasted_iota(i32, (_WIN, _CW), 0).astype(f32)

    def prep(n):
        r0 = pl.multiple_of(ints[0, 0, n], 8)
        c0 = ints[0, 1, n]
        gminx = bpf[0, n, 0]
        gminy = bpf[0, n, 1]
        gmaxx = bpf[0, n, 2]
        gmaxy = bpf[0, n, 3]
        a2 = bpf[0, n, 4]
        minwi = bpf[0, n, 5]
        maxwi = bpf[0, n, 6]
        minhi = bpf[0, n, 7]
        maxhi = bpf[0, n, 8]
        sl = pl.ds(r0, _WIN)
        iw = jnp.maximum(
            jnp.minimum(bmaxx[sl, :], gmaxx) - jnp.maximum(bminx[sl, :], gminx), 0.0)
        ih = jnp.maximum(
            jnp.minimum(bmaxy[sl, :], gmaxy) - jnp.maximum(bminy[sl, :], gminy), 0.0)
        inter = iw * ih
        union = a1s[sl, :] + a2 - inter
        iou = inter / jnp.maximum(union, 1e-06)
        iouc = pltpu.roll(iou, -c0, axis=1)[:, :_CW]
        jg = jgc + c0.astype(f32)
        ig = ig0 + r0.astype(f32)
        ff = ((jg >= minwi) & (jg <= maxwi) & (ig >= minhi) & (ig <= maxhi))
        iouf = jnp.where(ff, iouc, 0.0)
        s = jnp.sum(iouf)
        dkf = jnp.ceil(jnp.maximum(s, 1.0))
        xi = lax.bitcast_convert_type(iouf, i32)
        return (xi, dkf.astype(i32), dkf, r0, c0)

    def group_body(g, carry):
        sts = [prep(_K * g + j) for j in range(_K)]
        bounds = [(jnp.int32(0), jnp.int32(0x3F800000))] * _K
        for _ in range(30):
            new_bounds = []
            for (xi, dki, _, _, _), (lo, hi) in zip(sts, bounds):
                mid = (lo + hi) >> 1
                cnt = jnp.sum((xi > mid).astype(i32))
                le = cnt <= dki
                new_bounds.append((jnp.where(le, lo, mid + 1),
                                   jnp.where(le, mid, hi)))
            bounds = new_bounds
        for j in range(_K):
            xi, _, dkf, r0, c0 = sts[j]
            thr_bits = bounds[j][1]
            survc = jnp.where(xi > thr_bits,
                              lax.bitcast_convert_type(xi, f32), 0.0)
            surv = pltpu.roll(
                jnp.concatenate([survc, jnp.zeros((_WIN, _W - _CW), f32)],
                                axis=1), c0, axis=1)
            n = _K * g + j
            sl = pl.ds(r0, _WIN)
            cur = best[sl, :]
            mwin = surv > cur
            best[sl, :] = jnp.where(mwin, surv, cur)
            lam = jnp.sqrt(1.0 / dkf)
            for ref, ci, val in ((out_cls, 0, bpf[0, n, 13]),
                                 (out_cls, 1, bpf[0, n, 14]),
                                 (out_loc, 0, bpf[0, n, 9]),
                                 (out_loc, 1, bpf[0, n, 10]),
                                 (out_loc, 2, bpf[0, n, 11]),
                                 (out_loc, 3, bpf[0, n, 12]),
                                 (out_loc, 4, lam)):
                curp = ref[0, ci, sl, :]
                ref[0, ci, sl, :] = jnp.where(mwin, val, curp)
        return carry

    lax.fori_loop(0, _NB // _K, group_body, 0)


_SCRD = 56
_SCR = 49
_SCC = 64


def _sc_ceil_pos_v(xv):
    cv = jnp.maximum(xv, 1.0)
    iv = cv.astype(jnp.int32)
    fv = iv.astype(jnp.float32)
    dkiv = iv + jnp.where(fv < cv, 1, 0)
    return dkiv, dkiv.astype(jnp.float32)


def _sc_splat_sum(v, lanes):
    for sh in (8, 4, 2, 1):
        idx = (lanes + sh) & 15
        v = v + v.at[idx].get(mode="promise_in_bounds")
    return v


def _sc_rowstarts(minwi, minhi):
    ri = jnp.minimum(jnp.maximum(minhi, 0.0), 255.0).astype(jnp.int32)
    r0 = jnp.minimum((ri // 8) * 8, _H - _SCRD)
    mw = jnp.minimum(jnp.maximum(minwi, 0.0), 255.0).astype(jnp.int32)
    cc0 = jnp.minimum((mw // 16) * 16, _W - _SCC)
    return r0, cc0


def _sc_body(in1, bpf, out_cls, out_loc, out_thr,
             bpf_v, pw, xi, red_v, cnt_ref, lo_ref, hi_ref,
             sp, planes, stage_v, thrlam_v, shared):
    f32 = jnp.float32
    i32 = jnp.int32
    bl = lax.axis_index("c")
    sid = lax.axis_index("s")
    lanes = lax.broadcasted_iota(i32, (16,), 0)
    lanesf = lanes.astype(f32)
    zidx = jnp.zeros((16,), i32)

    pltpu.sync_copy(bpf.at[bl], bpf_v)

    def box_round(rnd, carry):
        n = 16 * rnd + sid

        @pl.when(n < _NB)
        def _():
            rowv = bpf_v[n, :]
            gminx = rowv[0]
            gminy = rowv[1]
            gmaxx = rowv[2]
            gmaxy = rowv[3]
            a2 = rowv[4]
            minwi = rowv[5]
            maxwi = rowv[6]
            minhi = rowv[7]
            maxhi = rowv[8]
            r0, cc0 = _sc_rowstarts(minwi, minhi)
            for ch in range(4):
                pltpu.sync_copy(
                    in1.at[pl.ds((bl * 4 + ch) * (_H * _W) + r0 * _W,
                                 _SCRD * _W)], pw[ch])
            cc0f = cc0.astype(f32)
            r0f = r0.astype(f32)

            red_v[:] = jnp.zeros((16,), f32)

            def row_fn(rr, acc):
                gy = (r0f + rr.astype(f32)) * 2.0 + 1.0
                rb = rr * _W + cc0
                for k in range(4):
                    p0c = pw[0][pl.ds(rb + 16 * k, 16)]
                    p1c = pw[1][pl.ds(rb + 16 * k, 16)]
                    p2c = pw[2][pl.ds(rb + 16 * k, 16)]
                    p3c = pw[3][pl.ds(rb + 16 * k, 16)]
                    jv = cc0f + (16 * k) + lanesf
                    gx = jv * 2.0 + 1.0
                    x1 = p0c * _SCALE + gx
                    y1 = p1c * _SCALE + gy
                    x2 = p2c * _SCALE + gx
                    y2 = p3c * _SCALE + gy
                    w = x2 - x1
                    h = y2 - y1
                    cx = x1 + w / 2.0
                    cy = y1 + h / 2.0
                    iw = jnp.maximum(
                        jnp.minimum(cx + w / 2.0, gmaxx)
                        - jnp.maximum(cx - w / 2.0, gminx), 0.0)
                    ihh = jnp.maximum(
                        jnp.minimum(cy + h / 2.0, gmaxy)
                        - jnp.maximum(cy - h / 2.0, gminy), 0.0)
                    inter = iw * ihh
                    union = w * h + a2 - inter
                    iou = inter / jnp.maximum(union, 1e-06)
                    igv = jnp.broadcast_to(r0f + rr.astype(f32), (16,))
                    m1 = jnp.where(jv >= minwi, 1.0, 0.0)
                    m2 = jnp.where(jv <= maxwi, 1.0, 0.0)
                    m3 = jnp.where(igv >= minhi, 1.0, 0.0)
                    m4 = jnp.where(igv <= maxhi, 1.0, 0.0)
                    iouf = iou * (m1 * m2) * (m3 * m4)
                    xi[pl.ds(64 * rr + 16 * k, 16)] = lax.bitcast_convert_type(iouf, i32)
                    plsc.addupdate(red_v.at[:], iouf)
                return acc

            lax.fori_loop(0, _SCR, row_fn, 0)
            sv = _sc_splat_sum(red_v[:], lanes)
            dki_v, dkf_v = _sc_ceil_pos_v(sv)

            lo_ref[:] = jnp.zeros((16,), i32)
            hi_ref[:] = jnp.full((16,), 0x3F800000, i32)

            def bs_step(it, carry2):
                lo = lo_ref[:]
                hi = hi_ref[:]
                mid = (lo + hi) >> 1
                cnt_ref[:] = jnp.zeros((16,), i32)

                def cnt_fn(j, c2):
                    m = xi[pl.ds(16 * j, 16)] > mid
                    plsc.addupdate(cnt_ref.at[:], jnp.where(m, 1, 0))
                    return c2

                lax.fori_loop(0, _SCR * 4, cnt_fn, 0)
                cntv = _sc_splat_sum(cnt_ref[:], lanes)
                le = cntv <= dki_v
                lo_ref[:] = jnp.where(le, lo, mid + 1)
                hi_ref[:] = jnp.where(le, mid, hi)
                return carry2

            lax.fori_loop(0, 30, bs_step, 0)
            thr_v = lax.bitcast_convert_type(hi_ref[:], f32)
            rv = 1.0 / dkf_v
            y = lax.bitcast_convert_type(
                0x5F3759DF - (lax.bitcast_convert_type(rv, i32) >> 1), f32)
            for _i in range(4):
                y = y * (1.5 - 0.5 * rv * y * y)
            sq = rv * y
            sq = 0.5 * (sq + rv / sq)
            stage_v[:] = jnp.where(lanes == 0, thr_v,
                                   jnp.where(lanes == 1, sq, 0.0))
            pltpu.sync_copy(stage_v,
                            out_thr.at[pl.ds(bl * 1024 + 16 * n, 16)])
        return carry

    lax.fori_loop(0, 4, box_round, 0)
    plsc.subcore_barrier()

    pltpu.sync_copy(out_thr.at[pl.ds(bl * 1024, 1024)], thrlam_v)
    best, c0p, c1p, l0p, l1p, l2p, l3p, l4p = planes


    for q in range(2):
        st = 2 * sid + q
        r8 = 8 * st
        for ch in range(4):
            pltpu.sync_copy(
                in1.at[pl.ds((bl * 4 + ch) * (_H * _W) + r8 * _W, 8 * _W)],
                sp[ch])

        def init_fn(k, carry):
            cs = pl.ds(16 * k, 16)
            best[cs] = jnp.zeros((16,), f32)
            c0p[cs] = jnp.ones((16,), f32)
            c1p[cs] = jnp.zeros((16,), f32)
            for p in (l0p, l1p, l2p, l3p, l4p):
                p[cs] = jnp.ones((16,), f32)
            return carry

        lax.fori_loop(0, 128, init_fn, 0)
        r8f = jnp.float32(8.0) * st.astype(f32)

        def box_fn(n, carry):
            rowv = bpf_v[n, :]
            minwi = rowv[5]
            maxwi = rowv[6]
            minhi = rowv[7]
            maxhi = rowv[8]
            hit = (minhi <= r8f + 7.0) & (maxhi >= r8f)

            @pl.when(hit)
            def _():
                gminx = rowv[0]
                gminy = rowv[1]
                gmaxx = rowv[2]
                gmaxy = rowv[3]
                a2 = rowv[4]
                gcx = rowv[9]
                gcy = rowv[10]
                gbw = rowv[11]
                gbh = rowv[12]
                cb0 = rowv[13]
                cb1 = rowv[14]
                tlv = thrlam_v[pl.ds(16 * n, 16)]
                thr = tlv[0]
                lam = tlv[1]
                _, cc0 = _sc_rowstarts(minwi, minhi)
                cc0f = cc0.astype(f32)
                for r in range(8):
                    igf = r8f + float(r)
                    igv = jnp.broadcast_to(igf, (16,))
                    gy = igf * 2.0 + 1.0
                    rb = r * _W + cc0
                    for k in range(4):
                        cs = pl.ds(rb + 16 * k, 16)
                        p0c = sp[0][cs]
                        p1c = sp[1][cs]
                        p2c = sp[2][cs]
                        p3c = sp[3][cs]
                        jv = cc0f + (16 * k) + lanesf
                        gx = jv * 2.0 + 1.0
                        x1 = p0c * _SCALE + gx
                        y1 = p1c * _SCALE + gy
                        x2 = p2c * _SCALE + gx
                        y2 = p3c * _SCALE + gy
                        w = x2 - x1
                        h = y2 - y1
                        cx = x1 + w / 2.0
                        cy = y1 + h / 2.0
                        iw = jnp.maximum(
                            jnp.minimum(cx + w / 2.0, gmaxx)
                            - jnp.maximum(cx - w / 2.0, gminx), 0.0)
                        ihh = jnp.maximum(
                            jnp.minimum(cy + h / 2.0, gmaxy)
                            - jnp.maximum(cy - h / 2.0, gminy), 0.0)
                        inter = iw * ihh
                        union = w * h + a2 - inter
                        iou = inter / jnp.maximum(union, 1e-06)
                        m1 = jnp.where(jv >= minwi, 1.0, 0.0)
                        m2 = jnp.where(jv <= maxwi, 1.0, 0.0)
                        m3 = jnp.where(igv >= minhi, 1.0, 0.0)
                        m4 = jnp.where(igv <= maxhi, 1.0, 0.0)
                        iouf = iou * (m1 * m2) * (m3 * m4)
                        val = jnp.where(iouf > thr, iouf, 0.0)
                        bc = best[cs]
                        m = val > bc
                        best[cs] = jnp.where(m, val, bc)
                        for ref, pv in ((c0p, cb0), (c1p, cb1), (l0p, gcx),
                                        (l1p, gcy), (l2p, gbw), (l3p, gbh),
                                        (l4p, lam)):
                            ref[cs] = jnp.where(m, pv, ref[cs])
            return carry

        lax.fori_loop(0, _NB, box_fn, 0)
        hw = _H * _W
        pltpu.sync_copy(c0p, out_cls.at[pl.ds((bl * 2 + 0) * hw + r8 * _W,
                                              8 * _W)])
        pltpu.sync_copy(c1p, out_cls.at[pl.ds((bl * 2 + 1) * hw + r8 * _W,
                                              8 * _W)])
        for ci, ref in enumerate((l0p, l1p, l2p, l3p, l4p)):
            pltpu.sync_copy(ref, out_loc.at[pl.ds((bl * 5 + ci) * hw + r8 * _W,
                                                  8 * _W)])


def _sc_half(in1_sc, bpf_sc):
    f32 = jnp.float32
    nb2 = bpf_sc.shape[0]
    mesh = plsc.VectorSubcoreMesh(core_axis_name="c", subcore_axis_name="s")

    @functools.partial(
        pl.kernel,
        out_type=[jax.ShapeDtypeStruct((nb2 * 2 * _H * _W,), f32),
                  jax.ShapeDtypeStruct((nb2 * 5 * _H * _W,), f32),
                  jax.ShapeDtypeStruct((nb2 * 1024,), f32)],
        mesh=mesh,
        scratch_types=[
            pltpu.VMEM((_NB, 16), f32),
            [pltpu.VMEM((_SCRD * _W,), f32) for _ in range(4)],
            pltpu.VMEM((_SCR * _SCC,), jnp.int32),
            pltpu.VMEM((16,), f32),
            [pltpu.VMEM((16,), jnp.int32) for _ in range(3)],
            [pltpu.VMEM((8 * _W,), f32) for _ in range(4)],
            [pltpu.VMEM((8 * _W,), f32) for _ in range(8)],
            pltpu.VMEM((16,), f32),
            pltpu.VMEM((1024,), f32),
            pltpu.VMEM_SHARED((1024,), f32),
        ],
    )
    def k(in1, bpf, out_cls, out_loc, out_thr,
          bpf_v, pw, xi, red_v, ilh, sp, planes, stage_v, thrlam_v, shared):
        _sc_body(in1, bpf, out_cls, out_loc, out_thr,
                 bpf_v, pw, xi, red_v, ilh[0], ilh[1], ilh[2],
                 sp, planes, stage_v, thrlam_v, shared)

    return k(in1_sc, bpf_sc)[:2]


def kernel(input0, input1, bboxes_bs):
    del input0
    f32 = jnp.float32
    bs = input1.shape[0]
    b = bboxes_bs.astype(f32)
    bw = b[:, :, 2] - b[:, :, 0]
    bh = b[:, :, 3] - b[:, :, 1]
    cx = b[:, :, 0] + bw / 2.0
    cy = b[:, :, 1] + bh / 2.0
    gminx = cx - bw / 2.0
    gmaxx = cx + bw / 2.0
    gminy = cy - bh / 2.0
    gmaxy = cy + bh / 2.0
    a2 = bw * bh
    valid = (bw * bh) > 0.0
    big = jnp.float32(1e9)
    minwi = jnp.floor(jnp.maximum(gminx * _W / _MS - 0.5, 0.0))
    maxwi = jnp.ceil(jnp.minimum(gmaxx * _W / _MS - 0.5, _W - 1.0))
    minhi = jnp.floor(jnp.maximum(gminy * _H / _MS - 0.5, 0.0))
    maxhi = jnp.ceil(jnp.minimum(gmaxy * _H / _MS - 0.5, _H - 1.0))
    minwi = jnp.where(valid, minwi, big)
    minhi = jnp.where(valid, minhi, big)
    cls_id = jnp.clip(b[:, :, 4].astype(jnp.int32), 0, 1)
    c0 = (cls_id == 0).astype(f32)
    c1 = (cls_id == 1).astype(f32)
    bpf = jnp.stack([gminx, gminy, gmaxx, gmaxy, a2,
                     minwi, maxwi, minhi, maxhi,
                     cx, cy, bw, bh, c0, c1,
                     jnp.zeros_like(c0)], axis=-1)
    rows0 = jnp.minimum((jnp.maximum(minhi, 0.0).astype(jnp.int32) // 8) * 8,
                        _H - _WIN)
    cols0 = jnp.clip((jnp.clip(minwi, 0.0, 255.0).astype(jnp.int32) // 16) * 16,
                     0, _W - 1)
    ints = jnp.stack([rows0, cols0], axis=1)

    ntc = 2
    grid = (ntc,)
    out_cls, out_loc = pl.pallas_call(
        _target_kernel,
        grid=grid,
        in_specs=[
            pl.BlockSpec((1, 4, _H, _W), lambda i: (i, 0, 0, 0)),
            pl.BlockSpec((1, _NB, 16), lambda i: (i, 0, 0),
                         memory_space=pltpu.SMEM),
            pl.BlockSpec((1, 2, _NB), lambda i: (i, 0, 0),
                         memory_space=pltpu.SMEM),
        ],
        out_specs=[
            pl.BlockSpec((1, 2, _H, _W), lambda i: (i, 0, 0, 0)),
            pl.BlockSpec((1, 5, _H, _W), lambda i: (i, 0, 0, 0)),
        ],
        out_shape=[
            jax.ShapeDtypeStruct((ntc, 2, _H, _W), f32),
            jax.ShapeDtypeStruct((ntc, 5, _H, _W), f32),
        ],
        scratch_shapes=[pltpu.VMEM((_H, _W), f32) for _ in range(6)],
    )(input1[:ntc], bpf[:ntc], ints[:ntc])

    sc_cls, sc_loc = _sc_half(
        input1[ntc:].reshape(-1), bpf[ntc:])
    sc_cls = sc_cls.reshape(bs - ntc, 2, _H, _W)
    sc_loc = sc_loc.reshape(bs - ntc, 5, _H, _W)
    out_cls = jnp.concatenate([out_cls, sc_cls], axis=0)
    out_loc = jnp.concatenate([out_loc, sc_loc], axis=0)

    class_map = jnp.transpose(out_cls, (0, 2, 3, 1))
    loc5 = jnp.transpose(out_loc, (0, 2, 3, 1))
    ones = jnp.ones((bs, _H, _W, 1), f32)
    loc_map = jnp.concatenate([loc5, ones], axis=-1)
    return class_map, loc_map
